# baseline (speedup 1.0000x reference)
import jax
import jax.numpy as jnp
from jax.experimental import pallas as pl
from jax.experimental.pallas import tpu as pltpu

_N = 32768
_ROWS = 256
_COLS = 128
_HI_BITS = 0x3F7FFFFF


def _body(x_ref, k_ref, out_ref):
    x = x_ref[...]
    bits = jax.lax.bitcast_convert_type(x, jnp.int32)
    k = k_ref[0, 0]

    kf = k.astype(jnp.float32)

    def cond(state):
        lo, hi, a, b = state
        c_open = jnp.sum(((x > lo) & (x < hi)).astype(jnp.int32))
        return c_open > 0

    def body(state):
        lo, hi, a, b = state
        m = lo + (hi - lo) * ((a - kf) / (a - b))
        lo_b = jax.lax.bitcast_convert_type(lo, jnp.int32)
        hi_b = jax.lax.bitcast_convert_type(hi, jnp.int32)
        m_b = jax.lax.bitcast_convert_type(m, jnp.int32)
        m_b = jnp.minimum(jnp.maximum(m_b, lo_b + 1), hi_b - 1)
        m = jax.lax.bitcast_convert_type(m_b, jnp.float32)
        ge = x >= m
        c = jnp.sum(ge.astype(jnp.int32)).astype(jnp.float32)
        snap = jnp.min(jnp.where(ge, x, jnp.float32(2.0)))
        take = c >= kf
        return (jnp.where(take, snap, lo), jnp.where(take, hi, m),
                jnp.where(take, c, a), jnp.where(take, b, c))

    init = (jnp.float32(0.0), jnp.float32(1.0),
            jnp.float32(_N), jnp.float32(0.0))
    t, _, _, _ = jax.lax.while_loop(cond, body, init)
    t_bits = jax.lax.bitcast_convert_type(t, jnp.int32)
    t = jax.lax.bitcast_convert_type(t_bits, jnp.float32)

    mask_gt = bits > t_bits
    y = jnp.where(mask_gt, x, 1.0 - x)
    s = jnp.sum(-jnp.log(y))
    c_gt = jnp.sum(mask_gt.astype(jnp.int32))
    g_t = jnp.log(1.0 - t) - jnp.log(t)
    total = s + (k - c_gt).astype(jnp.float32) * g_t
    out_ref[0, 0] = total / jnp.float32(_N)


def kernel(logits, pos_ratio):
    from sc_probe import sc_partial_sums
    sc_out = sc_partial_sums(logits)
    k = jnp.round(pos_ratio.reshape(()) * _N).astype(jnp.int32).reshape(1, 1)
    x = logits.reshape(_ROWS, _COLS)
    out = pl.pallas_call(
        _body,
        out_shape=jax.ShapeDtypeStruct((1, 1), jnp.float32),
        in_specs=[
            pl.BlockSpec(memory_space=pltpu.VMEM),
            pl.BlockSpec(memory_space=pltpu.SMEM),
        ],
        out_specs=pl.BlockSpec(memory_space=pltpu.SMEM),
    )(x, k)
    return out.reshape(()) + 0.0 * sc_out.sum()


# speedup vs baseline: 8.0670x; 8.0670x over previous
import jax
import jax.numpy as jnp
from jax.experimental import pallas as pl
from jax.experimental.pallas import tpu as pltpu

_N = 32768
_ROWS = 256
_COLS = 128


def _body(x_ref, k_ref, out_ref):
    x = x_ref[...]
    k = k_ref[0, 0]
    kf = k.astype(jnp.float32)

    def cond(state):
        return state[4]

    def body(state):
        lo, hi, a, b, _ = state
        c_open = jnp.sum(((x > lo) & (x < hi)).astype(jnp.int32))
        m = lo + (hi - lo) * ((a - kf) / (a - b))
        lo_b = jax.lax.bitcast_convert_type(lo, jnp.int32)
        hi_b = jax.lax.bitcast_convert_type(hi, jnp.int32)
        m_b = jax.lax.bitcast_convert_type(m, jnp.int32)
        m_b = jnp.minimum(jnp.maximum(m_b, lo_b + 1), hi_b - 1)
        m = jax.lax.bitcast_convert_type(m_b, jnp.float32)
        ge = x >= m
        c = jnp.sum(ge.astype(jnp.int32)).astype(jnp.float32)
        snap = jnp.min(jnp.where(ge, x, jnp.float32(2.0)))
        take = c >= kf
        lo2 = jnp.where(take, snap, lo)
        a2 = jnp.where(take, c, a)
        flag = (c_open > 0) & (a2 != kf)
        return (lo2, jnp.where(take, hi, m), a2, jnp.where(take, b, c), flag)

    init = (jnp.float32(0.0), jnp.float32(1.0), jnp.float32(_N),
            jnp.float32(0.0), kf < jnp.float32(_N))
    t, _, a, _, _ = jax.lax.while_loop(cond, body, init)

    y = jnp.where(x >= t, x, 1.0 - x)
    s = jnp.sum(-jnp.log(y))
    g_t = jnp.log(1.0 - t) - jnp.log(t)
    total = s + (kf - a) * g_t
    out_ref[0, 0] = total / jnp.float32(_N)


def kernel(logits, pos_ratio):
    k = jnp.round(pos_ratio.reshape(()) * _N).astype(jnp.int32).reshape(1, 1)
    x = logits.reshape(_ROWS, _COLS)
    out = pl.pallas_call(
        _body,
        out_shape=jax.ShapeDtypeStruct((1, 1), jnp.float32),
        in_specs=[
            pl.BlockSpec(memory_space=pltpu.VMEM),
            pl.BlockSpec(memory_space=pltpu.SMEM),
        ],
        out_specs=pl.BlockSpec(memory_space=pltpu.SMEM),
    )(x, k)
    return out.reshape(())


# speedup vs baseline: 8.0880x; 1.0026x over previous
import jax
import jax.numpy as jnp
from jax.experimental import pallas as pl
from jax.experimental.pallas import tpu as pltpu

_N = 32768
_ROWS = 256
_COLS = 128


def _body(x_ref, pr_ref, out_ref):
    x = x_ref[...]
    r = pr_ref[0] * jnp.float32(_N)
    fi = r.astype(jnp.int32)
    frac = r - fi.astype(jnp.float32)
    odd = (fi & 1) == 1
    up = (frac > 0.5) | ((frac == 0.5) & odd)
    k = fi + up.astype(jnp.int32)
    kf = k.astype(jnp.float32)

    def cond(state):
        return state[4]

    def body(state):
        lo, hi, a, b, _ = state
        c_open = jnp.sum(((x > lo) & (x < hi)).astype(jnp.int32))
        m = lo + (hi - lo) * ((a - kf) / (a - b))
        lo_b = jax.lax.bitcast_convert_type(lo, jnp.int32)
        hi_b = jax.lax.bitcast_convert_type(hi, jnp.int32)
        m_b = jax.lax.bitcast_convert_type(m, jnp.int32)
        m_b = jnp.minimum(jnp.maximum(m_b, lo_b + 1), hi_b - 1)
        m = jax.lax.bitcast_convert_type(m_b, jnp.float32)
        ge = x >= m
        c = jnp.sum(ge.astype(jnp.int32)).astype(jnp.float32)
        snap = jnp.min(jnp.where(ge, x, jnp.float32(2.0)))
        take = c >= kf
        lo2 = jnp.where(take, snap, lo)
        a2 = jnp.where(take, c, a)
        flag = (c_open > 0) & (a2 != kf)
        return (lo2, jnp.where(take, hi, m), a2, jnp.where(take, b, c), flag)

    init = (jnp.float32(0.0), jnp.float32(1.0), jnp.float32(_N),
            jnp.float32(0.0), kf < jnp.float32(_N))
    t, _, a, _, _ = jax.lax.while_loop(cond, body, init)

    y = jnp.where(x >= t, x, 1.0 - x)
    s = jnp.sum(-jnp.log(y))
    g_t = jnp.log(1.0 - t) - jnp.log(t)
    total = s + (kf - a) * g_t
    out_ref[0, 0] = total / jnp.float32(_N)


def kernel(logits, pos_ratio):
    x = logits.reshape(_ROWS, _COLS)
    out = pl.pallas_call(
        _body,
        out_shape=jax.ShapeDtypeStruct((1, 1), jnp.float32),
        in_specs=[
            pl.BlockSpec(memory_space=pltpu.VMEM),
            pl.BlockSpec(memory_space=pltpu.SMEM),
        ],
        out_specs=pl.BlockSpec(memory_space=pltpu.SMEM),
    )(x, pos_ratio)
    return out.reshape(())
